# baseline (speedup 1.0000x reference)
import functools

import jax
import jax.numpy as jnp
from jax import lax
from jax.experimental import pallas as pl
from jax.experimental.pallas import tpu as pltpu
from jax.experimental.pallas import tpu_sc as plsc

NC = 2
NS = 16
NW = NC * NS
LANES = 16
PACK = 4
CHUNK = 128


@functools.partial(jax.jit, static_argnames=("B", "C", "E"))
def _anime2vec_sc(target, ctx_flat, ttab_packed, ctab_packed, *, B, C, E):
    BPW = B // NW
    RPW = BPW * C
    NP = 8
    PB = BPW // NP
    PR = RPW // NP
    GPP = PR // LANES
    mesh = plsc.VectorSubcoreMesh(core_axis_name="c", subcore_axis_name="s")

    @functools.partial(
        pl.kernel,
        mesh=mesh,
        out_type=jax.ShapeDtypeStruct((B * C,), jnp.float32),
        scratch_types=[
            pltpu.VMEM((BPW,), jnp.int32),
            pltpu.VMEM((RPW,), jnp.int32),
            pltpu.VMEM((BPW,), jnp.int32),
            pltpu.VMEM((RPW,), jnp.int32),
            pltpu.VMEM((2, PB, PACK * E), jnp.float32),
            pltpu.VMEM((2, PR, PACK * E), jnp.float32),
            pltpu.VMEM((RPW,), jnp.float32),
            pltpu.SemaphoreType.DMA,
            pltpu.SemaphoreType.DMA,
        ],
        compiler_params=pltpu.CompilerParams(needs_layout_passes=False),
    )
    def k(tgt_hbm, ctx_hbm, ttab_hbm, ctab_hbm, out_hbm,
          tgt_idx, ctx_idx, tgt_rid, ctx_rid, tbuf, cbuf, out_v,
          sem_a, sem_b):
        wid = lax.axis_index("s") * NC + lax.axis_index("c")
        b0 = wid * BPW
        r0 = wid * RPW

        pltpu.sync_copy(tgt_hbm.at[pl.ds(b0, BPW)], tgt_idx)
        pltpu.sync_copy(ctx_hbm.at[pl.ds(r0, RPW)], ctx_idx)

        @pl.loop(0, BPW // LANES)
        def _(j):
            tgt_rid[pl.ds(j * LANES, LANES)] = (
                tgt_idx[pl.ds(j * LANES, LANES)] >> 2)

        @pl.loop(0, RPW // LANES)
        def _(j):
            ctx_rid[pl.ds(j * LANES, LANES)] = (
                ctx_idx[pl.ds(j * LANES, LANES)] >> 2)

        sems = (sem_a, sem_b)

        def fire(p):
            slot = p % 2
            sem = sems[slot]
            hs = [pltpu.async_copy(
                ttab_hbm.at[tgt_rid.at[pl.ds(p * PB, PB)]],
                tbuf.at[slot], sem)]
            for j in range(0, PR, CHUNK):
                n = min(CHUNK, PR - j)
                hs.append(pltpu.async_copy(
                    ctab_hbm.at[ctx_rid.at[pl.ds(p * PR + j, n)]],
                    cbuf.at[slot, pl.ds(j, n)], sem))
            return hs

        iota16 = lax.iota(jnp.int32, 16)
        pend = {0: fire(0), 1: fire(1)}

        for p in range(NP):
            slot = p % 2
            for h in pend.pop(p):
                h.wait()

            @pl.loop(0, GPP)
            def _(g, p=p, slot=slot):
                rl = p * PR + g * LANES
                civ = ctx_idx[pl.ds(rl, LANES)]
                c_off = (civ & 3) << 5
                rvec = rl + iota16
                bvec = rvec // C
                tb = bvec - p * PB
                tiv = plsc.load_gather(tgt_idx, [bvec])
                t_off = (tiv & 3) << 5
                pos = g * LANES + iota16
                acc = jnp.zeros((LANES,), jnp.float32)
                for e in range(E):
                    t = plsc.load_gather(tbuf, [jnp.full((LANES,), slot,
                                                         jnp.int32),
                                                tb, t_off + e])
                    c = plsc.load_gather(cbuf, [jnp.full((LANES,), slot,
                                                         jnp.int32),
                                                pos, c_off + e])
                    acc = acc + t * c
                out_v[pl.ds(rl, LANES)] = acc

            if p + 2 < NP:
                pend[p + 2] = fire(p + 2)

        pltpu.sync_copy(out_v, out_hbm.at[pl.ds(r0, RPW)])

    return k(target, ctx_flat, ttab_packed, ctab_packed)


def kernel(target, context, target_table, context_table):
    B, = target.shape
    _, C = context.shape
    V, E = target_table.shape
    out_flat = _anime2vec_sc(
        target,
        context.reshape(B * C),
        target_table.reshape(V // PACK, PACK * E),
        context_table.reshape(V // PACK, PACK * E),
        B=B, C=C, E=E)
    return out_flat.reshape(B, C)


# speedup vs baseline: 1.6206x; 1.6206x over previous
import functools

import jax
import jax.numpy as jnp
from jax import lax
from jax.experimental import pallas as pl
from jax.experimental.pallas import tpu as pltpu
from jax.experimental.pallas import tpu_sc as plsc

NC = 2
NS = 16
NW = NC * NS
LANES = 16
PACK = 4
CHUNK = 128
VBLK = 8192
SUB = VBLK // PACK


def _pack_kernel(t_ref, c_ref, to_ref, co_ref):
    for q in range(PACK):
        to_ref[:, q * 32:(q + 1) * 32] = t_ref[:, q * SUB:(q + 1) * SUB].T
        co_ref[:, q * 32:(q + 1) * 32] = c_ref[:, q * SUB:(q + 1) * SUB].T


def _pack_tables(ttab_t, ctab_t):
    E, V = ttab_t.shape
    grid = (V + VBLK - 1) // VBLK
    rows = grid * SUB
    return pl.pallas_call(
        _pack_kernel,
        grid=(grid,),
        in_specs=[
            pl.BlockSpec((E, VBLK), lambda i: (0, i)),
            pl.BlockSpec((E, VBLK), lambda i: (0, i)),
        ],
        out_specs=[
            pl.BlockSpec((SUB, PACK * E), lambda i: (i, 0)),
            pl.BlockSpec((SUB, PACK * E), lambda i: (i, 0)),
        ],
        out_shape=[
            jax.ShapeDtypeStruct((rows, PACK * E), jnp.float32),
            jax.ShapeDtypeStruct((rows, PACK * E), jnp.float32),
        ],
    )(ttab_t, ctab_t)


def _packed_row(v):
    return ((v >> 13) << 11) | (v & 2047)


@functools.partial(jax.jit, static_argnames=("B", "C", "E"))
def _anime2vec_sc(target, ctx_flat, ttab_t, ctab_t, *, B, C, E):
    ttab_packed, ctab_packed = _pack_tables(ttab_t, ctab_t)

    BPW = B // NW
    RPW = BPW * C
    NP = 8
    PB = BPW // NP
    PR = RPW // NP
    GPP = PR // LANES
    mesh = plsc.VectorSubcoreMesh(core_axis_name="c", subcore_axis_name="s")

    @functools.partial(
        pl.kernel,
        mesh=mesh,
        out_type=jax.ShapeDtypeStruct((B * C,), jnp.float32),
        scratch_types=[
            pltpu.VMEM((BPW,), jnp.int32),
            pltpu.VMEM((RPW,), jnp.int32),
            pltpu.VMEM((BPW,), jnp.int32),
            pltpu.VMEM((RPW,), jnp.int32),
            pltpu.VMEM((2, PB, PACK * E), jnp.float32),
            pltpu.VMEM((2, PR, PACK * E), jnp.float32),
            pltpu.VMEM((RPW,), jnp.float32),
            pltpu.SemaphoreType.DMA,
            pltpu.SemaphoreType.DMA,
        ],
        compiler_params=pltpu.CompilerParams(needs_layout_passes=False),
    )
    def k(tgt_hbm, ctx_hbm, ttab_hbm, ctab_hbm, out_hbm,
          tgt_idx, ctx_idx, tgt_rid, ctx_rid, tbuf, cbuf, out_v,
          sem_a, sem_b):
        wid = lax.axis_index("s") * NC + lax.axis_index("c")
        b0 = wid * BPW
        r0 = wid * RPW

        pltpu.sync_copy(tgt_hbm.at[pl.ds(b0, BPW)], tgt_idx)
        pltpu.sync_copy(ctx_hbm.at[pl.ds(r0, RPW)], ctx_idx)

        @pl.loop(0, BPW // LANES)
        def _(j):
            tgt_rid[pl.ds(j * LANES, LANES)] = (
                _packed_row(tgt_idx[pl.ds(j * LANES, LANES)]))

        @pl.loop(0, RPW // LANES)
        def _(j):
            ctx_rid[pl.ds(j * LANES, LANES)] = (
                _packed_row(ctx_idx[pl.ds(j * LANES, LANES)]))

        sems = (sem_a, sem_b)

        def fire(p):
            slot = p % 2
            sem = sems[slot]
            hs = [pltpu.async_copy(
                ttab_hbm.at[tgt_rid.at[pl.ds(p * PB, PB)]],
                tbuf.at[slot], sem)]
            for j in range(0, PR, CHUNK):
                n = min(CHUNK, PR - j)
                hs.append(pltpu.async_copy(
                    ctab_hbm.at[ctx_rid.at[pl.ds(p * PR + j, n)]],
                    cbuf.at[slot, pl.ds(j, n)], sem))
            return hs

        iota16 = lax.iota(jnp.int32, 16)
        pend = {0: fire(0), 1: fire(1)}

        for p in range(NP):
            slot = p % 2
            for h in pend.pop(p):
                h.wait()

            @pl.loop(0, GPP)
            def _(g, p=p, slot=slot):
                rl = p * PR + g * LANES
                civ = ctx_idx[pl.ds(rl, LANES)]
                c_off = ((civ >> 11) & 3) << 5
                rvec = rl + iota16
                bvec = rvec // C
                tb = bvec - p * PB
                tiv = plsc.load_gather(tgt_idx, [bvec])
                t_off = ((tiv >> 11) & 3) << 5
                pos = g * LANES + iota16
                acc = jnp.zeros((LANES,), jnp.float32)
                for e in range(E):
                    t = plsc.load_gather(tbuf, [jnp.full((LANES,), slot,
                                                         jnp.int32),
                                                tb, t_off + e])
                    c = plsc.load_gather(cbuf, [jnp.full((LANES,), slot,
                                                         jnp.int32),
                                                pos, c_off + e])
                    acc = acc + t * c
                out_v[pl.ds(rl, LANES)] = acc

            if p + 2 < NP:
                pend[p + 2] = fire(p + 2)

        pltpu.sync_copy(out_v, out_hbm.at[pl.ds(r0, RPW)])

    return k(target, ctx_flat, ttab_packed, ctab_packed)


def kernel(target, context, target_table, context_table):
    B, = target.shape
    _, C = context.shape
    _, E = target_table.shape
    out_flat = _anime2vec_sc(
        target,
        context.reshape(B * C),
        target_table.T,
        context_table.T,
        B=B, C=C, E=E)
    return out_flat.reshape(B, C)
